# Initial kernel scaffold; baseline (speedup 1.0000x reference)
import jax
import jax.numpy as jnp
from jax.experimental import pallas as pl


def kernel(h, edge_index, e, snorm_n, snorm_e, W_h, b_h, W_pre, b_pre, W_post, b_post):
    raise NotImplementedError("write your pallas kernel here")


# speedup vs baseline: 2.1790x; 2.1790x over previous
import functools

import jax
import jax.numpy as jnp
from jax import lax
from jax.experimental import pallas as pl
from jax.experimental.pallas import tpu as pltpu
from jax.experimental.pallas import tpu_sc as plsc

N = 10000
E = 320000
D = 128
L = 4
AVG_D_LOG = 3.4965

NW = 32
NRW = 3
NR = NW * NRW
R = 112
NPAD = NR * R
NTC = NPAD
C = 128
EPAD = E + 2 * C
SENT = 1 << 28

BM = 512
BM2 = 400

_f32 = jnp.float32
_i32 = jnp.int32


def _k0_body(h_ref, w_ref, b_ref, o_ref):
    o_ref[...] = jnp.dot(h_ref[...], w_ref[...],
                         preferred_element_type=_f32) + b_ref[...]


def _k0(h, w, b):
    return pl.pallas_call(
        _k0_body,
        grid=(20,),
        in_specs=[
            pl.BlockSpec((BM, D), lambda i: (i, 0)),
            pl.BlockSpec((D, D), lambda i: (0, 0)),
            pl.BlockSpec((1, D), lambda i: (0, 0)),
        ],
        out_specs=pl.BlockSpec((BM, D), lambda i: (i, 0)),
        out_shape=jax.ShapeDtypeStruct((N, D), _f32),
    )(h, w, b)


def _k1_body(x_ref, w_ref, b_ref, a_ref, b2_ref):
    x = x_ref[...]
    a_ref[...] = jnp.dot(x, w_ref[0], preferred_element_type=_f32) + b_ref[...]
    b2_ref[...] = jnp.dot(x, w_ref[1], preferred_element_type=_f32)


def _k1(x, wstk, b):
    return pl.pallas_call(
        _k1_body,
        grid=(NTC // BM,),
        in_specs=[
            pl.BlockSpec((BM, D), lambda i: (jnp.minimum(i, (N - 1) // BM), 0)),
            pl.BlockSpec((2, D, D), lambda i: (0, 0, 0)),
            pl.BlockSpec((1, D), lambda i: (0, 0)),
        ],
        out_specs=[
            pl.BlockSpec((BM, D), lambda i: (i, 0)),
            pl.BlockSpec((BM, D), lambda i: (i, 0)),
        ],
        out_shape=[
            jax.ShapeDtypeStruct((NTC, D), _f32),
            jax.ShapeDtypeStruct((NTC, D), _f32),
        ],
    )(x, wstk, b)


def _sc_body(a_hbm, b_hbm, src_hbm, dst_hbm, starts_hbm,
             osum, osq, omx, omn, ocnt,
             b2win, arows, idxb, dstb, cntb, startsv,
             ssum, ssq, smx, smn, sem):
    wid = lax.axis_index("s") * 2 + lax.axis_index("c")
    pltpu.sync_copy(starts_hbm, startsv)

    zero = jnp.zeros((16,), _f32)
    one = jnp.ones((16,), _f32)
    neg = jnp.full((16,), -1e30, _f32)
    pos = jnp.full((16,), 1e30, _f32)

    def _range(i, _):
        r = wid * NRW + i
        lo = r * R
        sv = startsv[pl.ds(r, 16)]
        s = sv[0]
        e = sv[1]
        pltpu.sync_copy(b_hbm.at[pl.ds(lo, R)], b2win)

        def _init(ii, _):
            for g in range(8):
                sl = pl.ds(g * 16, 16)
                ssum[ii, sl] = zero
                ssq[ii, sl] = zero
                smx[ii, sl] = neg
                smn[ii, sl] = pos
            cntb[ii, :] = zero
            return 0
        lax.fori_loop(0, R, _init, 0)

        s8 = pl.multiple_of((s // 8) * 8, 8)
        nchunks = (e - s8 + C - 1) // C

        def _chunk(k, _):
            c = pl.multiple_of(s8 + k * C, 8)
            pltpu.sync_copy(src_hbm.at[pl.ds(c, C)], idxb)
            pltpu.sync_copy(dst_hbm.at[pl.ds(c, C)], dstb)
            pltpu.async_copy(a_hbm.at[idxb], arows, sem).wait()

            def _edge16(t, _):
                dvec = dstb[pl.ds(t * 16, 16)] - lo
                for k in range(16):
                    w = dvec[k]

                    @pl.when((w >= 0) & (w < R))
                    def _():
                        j = t * 16 + k
                        cntb[w, :] = cntb[w, :] + one
                        for g in range(8):
                            sl = pl.ds(g * 16, 16)
                            m = jnp.maximum(
                                arows[j, sl] + b2win[w, sl], 0.0)
                            ssum[w, sl] = ssum[w, sl] + m
                            ssq[w, sl] = ssq[w, sl] + m * m
                            smx[w, sl] = jnp.maximum(smx[w, sl], m)
                            smn[w, sl] = jnp.minimum(smn[w, sl], m)
                return 0
            lax.fori_loop(0, C // 16, _edge16, 0)
            return 0
        lax.fori_loop(0, nchunks, _chunk, 0)

        pltpu.sync_copy(ssum, osum.at[pl.ds(lo, R)])
        pltpu.sync_copy(ssq, osq.at[pl.ds(lo, R)])
        pltpu.sync_copy(smx, omx.at[pl.ds(lo, R)])
        pltpu.sync_copy(smn, omn.at[pl.ds(lo, R)])
        pltpu.sync_copy(cntb, ocnt.at[r])
        return 0

    lax.fori_loop(0, NRW, _range, 0)


@functools.cache
def _sc_call():
    return functools.partial(
        pl.kernel,
        out_type=[
        jax.ShapeDtypeStruct((NPAD, D), _f32),
        jax.ShapeDtypeStruct((NPAD, D), _f32),
        jax.ShapeDtypeStruct((NPAD, D), _f32),
        jax.ShapeDtypeStruct((NPAD, D), _f32),
            jax.ShapeDtypeStruct((NR, R, 16), _f32),
        ],
        mesh=plsc.VectorSubcoreMesh(core_axis_name="c", subcore_axis_name="s"),
        scratch_types=[
            pltpu.VMEM((R, D), _f32),
            pltpu.VMEM((C, D), _f32),
            pltpu.VMEM((C,), _i32),
            pltpu.VMEM((C,), _i32),
            pltpu.VMEM((R, 16), _f32),
            pltpu.VMEM((NR + 16,), _i32),
            pltpu.VMEM((R, D), _f32),
            pltpu.VMEM((R, D), _f32),
            pltpu.VMEM((R, D), _f32),
            pltpu.VMEM((R, D), _f32),
            pltpu.SemaphoreType.DMA,
        ],
    )(_sc_body)


def _k2_body(x_ref, sm_ref, sq_ref, mx_ref, mn_ref, cnt_ref, sn_ref,
             wph_ref, w3_ref, b_ref, hn_ref, s1_ref, s2_ref):
    i = pl.program_id(0)
    cnt = cnt_ref[...]
    d = jnp.maximum(cnt, 1.0)
    inv = 1.0 / d
    mean = sm_ref[...] * inv
    var = jnp.maximum(sq_ref[...] * inv - mean * mean, 0.0)
    std = jnp.sqrt(var + 1e-5)
    has = cnt > 0.0
    mx = jnp.where(has, mx_ref[...], 0.0)
    mn = jnp.where(has, mn_ref[...], 0.0)
    agg = jnp.concatenate([mean, mx, mn, std], axis=1)
    logd = jnp.log(d + 1.0)
    amp = logd * (1.0 / AVG_D_LOG)
    att = AVG_D_LOG / logd
    hn = jnp.dot(x_ref[...], wph_ref[...], preferred_element_type=_f32)
    hn = hn + jnp.dot(agg, w3_ref[0], preferred_element_type=_f32)
    hn = hn + amp * jnp.dot(agg, w3_ref[1], preferred_element_type=_f32)
    hn = hn + att * jnp.dot(agg, w3_ref[2], preferred_element_type=_f32)
    hn = (hn + b_ref[...]) * sn_ref[...]
    hn_ref[...] = hn

    @pl.when(i == 0)
    def _():
        s1_ref[...] = jnp.zeros_like(s1_ref)
        s2_ref[...] = jnp.zeros_like(s2_ref)

    s1_ref[...] += jnp.sum(hn, axis=0, keepdims=True)
    s2_ref[...] += jnp.sum(hn * hn, axis=0, keepdims=True)


def _k2(x, sm, sq, mx, mn, cnt, sn, wph, w3, b):
    return pl.pallas_call(
        _k2_body,
        grid=(N // BM2,),
        in_specs=[
            pl.BlockSpec((BM2, D), lambda i: (i, 0)),
            pl.BlockSpec((BM2, D), lambda i: (i, 0)),
            pl.BlockSpec((BM2, D), lambda i: (i, 0)),
            pl.BlockSpec((BM2, D), lambda i: (i, 0)),
            pl.BlockSpec((BM2, D), lambda i: (i, 0)),
            pl.BlockSpec((BM2, 1), lambda i: (i, 0)),
            pl.BlockSpec((BM2, 1), lambda i: (i, 0)),
            pl.BlockSpec((D, D), lambda i: (0, 0)),
            pl.BlockSpec((3, 4 * D, D), lambda i: (0, 0, 0)),
            pl.BlockSpec((1, D), lambda i: (0, 0)),
        ],
        out_specs=[
            pl.BlockSpec((BM2, D), lambda i: (i, 0)),
            pl.BlockSpec((1, D), lambda i: (0, 0)),
            pl.BlockSpec((1, D), lambda i: (0, 0)),
        ],
        out_shape=[
            jax.ShapeDtypeStruct((N, D), _f32),
            jax.ShapeDtypeStruct((1, D), _f32),
            jax.ShapeDtypeStruct((1, D), _f32),
        ],
    )(x, sm, sq, mx, mn, cnt, sn, wph, w3, b)


def _k3_body(hn_ref, x_ref, s1_ref, s2_ref, o_ref):
    mu = s1_ref[...] * (1.0 / N)
    var = s2_ref[...] * (1.0 / N) - mu * mu
    inv = lax.rsqrt(var + 1e-5)
    o_ref[...] = x_ref[...] + jnp.maximum((hn_ref[...] - mu) * inv, 0.0)


def _k3(hn, x, s1, s2):
    return pl.pallas_call(
        _k3_body,
        grid=(N // BM2,),
        in_specs=[
            pl.BlockSpec((BM2, D), lambda i: (i, 0)),
            pl.BlockSpec((BM2, D), lambda i: (i, 0)),
            pl.BlockSpec((1, D), lambda i: (0, 0)),
            pl.BlockSpec((1, D), lambda i: (0, 0)),
        ],
        out_specs=pl.BlockSpec((BM2, D), lambda i: (i, 0)),
        out_shape=jax.ShapeDtypeStruct((N, D), _f32),
    )(hn, x, s1, s2)


def kernel(h, edge_index, e, snorm_n, snorm_e, W_h, b_h, W_pre, b_pre,
           W_post, b_post):
    del e, snorm_e
    src = edge_index[0].astype(_i32)
    dst = edge_index[1].astype(_i32)

    order = jnp.argsort(dst)
    dst_s = dst[order]
    src_s = src[order]
    src_p = jnp.concatenate([src_s, jnp.zeros((EPAD - E,), _i32)])
    dst_p = jnp.concatenate([dst_s, jnp.full((EPAD - E,), SENT, _i32)])
    bounds = jnp.searchsorted(
        dst_s, (jnp.arange(NR + 1, dtype=_i32) * R)).astype(_i32)
    starts = jnp.concatenate([bounds, jnp.full((NR + 16 - (NR + 1),), E, _i32)])

    x = _k0(h, W_h, b_h[None])
    for l in range(L):
        a, b2 = _k1(x, W_pre[l].reshape(2, D, D), b_pre[l][None])
        sm, sq, mx, mn, cnt = _sc_call()(a, b2, src_p, dst_p, starts)
        cnt_n = cnt[:, :, 0].reshape(NPAD)[:N, None]
        hn, s1, s2 = _k2(x, sm[:N], sq[:N], mx[:N], mn[:N], cnt_n, snorm_n,
                         W_post[l, :D], W_post[l, D:].reshape(3, 4 * D, D),
                         b_post[l][None])
        x = _k3(hn, x, s1, s2)
    return x


# speedup vs baseline: 4.3616x; 2.0017x over previous
import functools

import jax
import jax.numpy as jnp
from jax import lax
from jax.experimental import pallas as pl
from jax.experimental.pallas import tpu as pltpu
from jax.experimental.pallas import tpu_sc as plsc

N = 10000
E = 320000
D = 128
L = 4
AVG_D_LOG = 3.4965

NW = 32
NRW = 3
NR = NW * NRW
R = 112
NPAD = NR * R
NTC = NPAD
C = 128
EPAD = E + 2 * C
SENT = 1 << 28

BM = 512
BM2 = 400

_f32 = jnp.float32
_i32 = jnp.int32


def _k0_body(h_ref, w_ref, b_ref, o_ref):
    o_ref[...] = jnp.dot(h_ref[...], w_ref[...],
                         preferred_element_type=_f32) + b_ref[...]


def _k0(h, w, b):
    return pl.pallas_call(
        _k0_body,
        grid=(20,),
        in_specs=[
            pl.BlockSpec((BM, D), lambda i: (i, 0)),
            pl.BlockSpec((D, D), lambda i: (0, 0)),
            pl.BlockSpec((1, D), lambda i: (0, 0)),
        ],
        out_specs=pl.BlockSpec((BM, D), lambda i: (i, 0)),
        out_shape=jax.ShapeDtypeStruct((N, D), _f32),
    )(h, w, b)


def _k1_body(x_ref, w_ref, b_ref, a_ref, b2_ref):
    x = x_ref[...]
    a_ref[...] = jnp.dot(x, w_ref[0], preferred_element_type=_f32) + b_ref[...]
    b2_ref[...] = jnp.dot(x, w_ref[1], preferred_element_type=_f32)


def _k1(x, wstk, b):
    return pl.pallas_call(
        _k1_body,
        grid=(NTC // BM,),
        in_specs=[
            pl.BlockSpec((BM, D), lambda i: (jnp.minimum(i, (N - 1) // BM), 0)),
            pl.BlockSpec((2, D, D), lambda i: (0, 0, 0)),
            pl.BlockSpec((1, D), lambda i: (0, 0)),
        ],
        out_specs=[
            pl.BlockSpec((BM, D), lambda i: (i, 0)),
            pl.BlockSpec((BM, D), lambda i: (i, 0)),
        ],
        out_shape=[
            jax.ShapeDtypeStruct((NTC, D), _f32),
            jax.ShapeDtypeStruct((NTC, D), _f32),
        ],
    )(x, wstk, b)


def _sc_body(a_hbm, b_hbm, src_hbm, dst_hbm, starts_hbm,
             osum, osq, omx, omn, ocnt,
             b2win, arows, idxb, dstb, cntb, startsv,
             ssum, ssq, smx, smn, sem):
    wid = lax.axis_index("s") * 2 + lax.axis_index("c")
    pltpu.sync_copy(starts_hbm, startsv)

    zero = jnp.zeros((16,), _f32)
    one = jnp.ones((16,), _f32)
    neg = jnp.full((16,), -1e30, _f32)
    pos = jnp.full((16,), 1e30, _f32)

    G = 8

    def _range(i, _):
        r = wid * NRW + i
        lo = r * R
        sv = startsv[pl.ds(r, 16)]
        s = sv[0]
        e = sv[1]
        pltpu.sync_copy(b_hbm.at[pl.ds(lo, R)], b2win.at[pl.ds(0, R)])

        def _init(ii, _):
            for g in range(G):
                sl = pl.ds(g * 16, 16)
                ssum[ii, sl] = zero
                ssq[ii, sl] = zero
                smx[ii, sl] = neg
                smn[ii, sl] = pos
            cntb[ii, :] = zero
            return 0
        lax.fori_loop(0, R + 1, _init, 0)

        s8 = pl.multiple_of((s // 8) * 8, 8)
        nchunks = (e - s8 + C - 1) // C

        def _chunk(k, _):
            c = pl.multiple_of(s8 + k * C, 8)
            pltpu.sync_copy(src_hbm.at[pl.ds(c, C)], idxb)
            pltpu.sync_copy(dst_hbm.at[pl.ds(c, C)], dstb)
            pltpu.async_copy(a_hbm.at[idxb], arows, sem).wait()

            def _edge16(t, _):
                dvec = dstb[pl.ds(t * 16, 16)] - lo
                for k16 in range(16):
                    w = dvec[k16]
                    w2 = jnp.where((w >= 0) & (w < R), w, R)
                    j = t * 16 + k16
                    plsc.addupdate(cntb.at[w2, :], one)
                    for g in range(G):
                        sl = pl.ds(g * 16, 16)
                        m = jnp.maximum(arows[j, sl] + b2win[w2, sl], 0.0)
                        plsc.addupdate(ssum.at[w2, sl], m)
                        plsc.addupdate(ssq.at[w2, sl], m * m)
                        smx[w2, sl] = jnp.maximum(smx[w2, sl], m)
                        smn[w2, sl] = jnp.minimum(smn[w2, sl], m)
                return 0
            lax.fori_loop(0, C // 16, _edge16, 0)
            return 0
        lax.fori_loop(0, nchunks, _chunk, 0)

        pltpu.sync_copy(ssum.at[pl.ds(0, R)], osum.at[pl.ds(lo, R)])
        pltpu.sync_copy(ssq.at[pl.ds(0, R)], osq.at[pl.ds(lo, R)])
        pltpu.sync_copy(smx.at[pl.ds(0, R)], omx.at[pl.ds(lo, R)])
        pltpu.sync_copy(smn.at[pl.ds(0, R)], omn.at[pl.ds(lo, R)])
        pltpu.sync_copy(cntb.at[pl.ds(0, R)], ocnt.at[r])
        return 0

    lax.fori_loop(0, NRW, _range, 0)


@functools.cache
def _sc_call():
    return functools.partial(
        pl.kernel,
        out_type=[
        jax.ShapeDtypeStruct((NPAD, D), _f32),
        jax.ShapeDtypeStruct((NPAD, D), _f32),
        jax.ShapeDtypeStruct((NPAD, D), _f32),
        jax.ShapeDtypeStruct((NPAD, D), _f32),
            jax.ShapeDtypeStruct((NR, R, 16), _f32),
        ],
        mesh=plsc.VectorSubcoreMesh(core_axis_name="c", subcore_axis_name="s"),
        scratch_types=[
            pltpu.VMEM((R + 8, D), _f32),
            pltpu.VMEM((C, D), _f32),
            pltpu.VMEM((C,), _i32),
            pltpu.VMEM((C,), _i32),
            pltpu.VMEM((R + 8, 16), _f32),
            pltpu.VMEM((NR + 16,), _i32),
            pltpu.VMEM((R + 8, D), _f32),
            pltpu.VMEM((R + 8, D), _f32),
            pltpu.VMEM((R + 8, D), _f32),
            pltpu.VMEM((R + 8, D), _f32),
            pltpu.SemaphoreType.DMA,
        ],
    )(_sc_body)


def _k2_body(x_ref, sm_ref, sq_ref, mx_ref, mn_ref, cnt_ref, sn_ref,
             wph_ref, w3_ref, b_ref, hn_ref, s1_ref, s2_ref):
    i = pl.program_id(0)
    cnt = cnt_ref[...]
    d = jnp.maximum(cnt, 1.0)
    inv = 1.0 / d
    mean = sm_ref[...] * inv
    var = jnp.maximum(sq_ref[...] * inv - mean * mean, 0.0)
    std = jnp.sqrt(var + 1e-5)
    has = cnt > 0.0
    mx = jnp.where(has, mx_ref[...], 0.0)
    mn = jnp.where(has, mn_ref[...], 0.0)
    agg = jnp.concatenate([mean, mx, mn, std], axis=1)
    logd = jnp.log(d + 1.0)
    amp = logd * (1.0 / AVG_D_LOG)
    att = AVG_D_LOG / logd
    hn = jnp.dot(x_ref[...], wph_ref[...], preferred_element_type=_f32)
    hn = hn + jnp.dot(agg, w3_ref[0], preferred_element_type=_f32)
    hn = hn + amp * jnp.dot(agg, w3_ref[1], preferred_element_type=_f32)
    hn = hn + att * jnp.dot(agg, w3_ref[2], preferred_element_type=_f32)
    hn = (hn + b_ref[...]) * sn_ref[...]
    hn_ref[...] = hn

    @pl.when(i == 0)
    def _():
        s1_ref[...] = jnp.zeros_like(s1_ref)
        s2_ref[...] = jnp.zeros_like(s2_ref)

    s1_ref[...] += jnp.sum(hn, axis=0, keepdims=True)
    s2_ref[...] += jnp.sum(hn * hn, axis=0, keepdims=True)


def _k2(x, sm, sq, mx, mn, cnt, sn, wph, w3, b):
    return pl.pallas_call(
        _k2_body,
        grid=(N // BM2,),
        in_specs=[
            pl.BlockSpec((BM2, D), lambda i: (i, 0)),
            pl.BlockSpec((BM2, D), lambda i: (i, 0)),
            pl.BlockSpec((BM2, D), lambda i: (i, 0)),
            pl.BlockSpec((BM2, D), lambda i: (i, 0)),
            pl.BlockSpec((BM2, D), lambda i: (i, 0)),
            pl.BlockSpec((BM2, 1), lambda i: (i, 0)),
            pl.BlockSpec((BM2, 1), lambda i: (i, 0)),
            pl.BlockSpec((D, D), lambda i: (0, 0)),
            pl.BlockSpec((3, 4 * D, D), lambda i: (0, 0, 0)),
            pl.BlockSpec((1, D), lambda i: (0, 0)),
        ],
        out_specs=[
            pl.BlockSpec((BM2, D), lambda i: (i, 0)),
            pl.BlockSpec((1, D), lambda i: (0, 0)),
            pl.BlockSpec((1, D), lambda i: (0, 0)),
        ],
        out_shape=[
            jax.ShapeDtypeStruct((N, D), _f32),
            jax.ShapeDtypeStruct((1, D), _f32),
            jax.ShapeDtypeStruct((1, D), _f32),
        ],
    )(x, sm, sq, mx, mn, cnt, sn, wph, w3, b)


def _k3_body(hn_ref, x_ref, s1_ref, s2_ref, o_ref):
    mu = s1_ref[...] * (1.0 / N)
    var = s2_ref[...] * (1.0 / N) - mu * mu
    inv = lax.rsqrt(var + 1e-5)
    o_ref[...] = x_ref[...] + jnp.maximum((hn_ref[...] - mu) * inv, 0.0)


def _k3(hn, x, s1, s2):
    return pl.pallas_call(
        _k3_body,
        grid=(N // BM2,),
        in_specs=[
            pl.BlockSpec((BM2, D), lambda i: (i, 0)),
            pl.BlockSpec((BM2, D), lambda i: (i, 0)),
            pl.BlockSpec((1, D), lambda i: (0, 0)),
            pl.BlockSpec((1, D), lambda i: (0, 0)),
        ],
        out_specs=pl.BlockSpec((BM2, D), lambda i: (i, 0)),
        out_shape=jax.ShapeDtypeStruct((N, D), _f32),
    )(hn, x, s1, s2)


def kernel(h, edge_index, e, snorm_n, snorm_e, W_h, b_h, W_pre, b_pre,
           W_post, b_post):
    del e, snorm_e
    src = edge_index[0].astype(_i32)
    dst = edge_index[1].astype(_i32)

    order = jnp.argsort(dst)
    dst_s = dst[order]
    src_s = src[order]
    src_p = jnp.concatenate([src_s, jnp.zeros((EPAD - E,), _i32)])
    dst_p = jnp.concatenate([dst_s, jnp.full((EPAD - E,), SENT, _i32)])
    bounds = jnp.searchsorted(
        dst_s, (jnp.arange(NR + 1, dtype=_i32) * R)).astype(_i32)
    starts = jnp.concatenate([bounds, jnp.full((NR + 16 - (NR + 1),), E, _i32)])

    x = _k0(h, W_h, b_h[None])
    for l in range(L):
        a, b2 = _k1(x, W_pre[l].reshape(2, D, D), b_pre[l][None])
        sm, sq, mx, mn, cnt = _sc_call()(a, b2, src_p, dst_p, starts)
        cnt_n = cnt[:, :, 0].reshape(NPAD)[:N, None]
        hn, s1, s2 = _k2(x, sm[:N], sq[:N], mx[:N], mn[:N], cnt_n, snorm_n,
                         W_post[l, :D], W_post[l, D:].reshape(3, 4 * D, D),
                         b_post[l][None])
        x = _k3(hn, x, s1, s2)
    return x


# speedup vs baseline: 4.7278x; 1.0840x over previous
import functools

import jax
import jax.numpy as jnp
from jax import lax
from jax.experimental import pallas as pl
from jax.experimental.pallas import tpu as pltpu
from jax.experimental.pallas import tpu_sc as plsc

N = 10000
E = 320000
D = 128
L = 4
AVG_D_LOG = 3.4965

NW = 32
NRW = 3
NR = NW * NRW
R = 112
NPAD = NR * R
NTC = NPAD
C = 128
EPAD = E + 2 * C
SENT = 1 << 28

BM = 512
BM2 = 400

_f32 = jnp.float32
_i32 = jnp.int32


def _k0_body(h_ref, w_ref, b_ref, o_ref):
    o_ref[...] = jnp.dot(h_ref[...], w_ref[...],
                         preferred_element_type=_f32) + b_ref[...]


def _k0(h, w, b):
    return pl.pallas_call(
        _k0_body,
        grid=(20,),
        in_specs=[
            pl.BlockSpec((BM, D), lambda i: (i, 0)),
            pl.BlockSpec((D, D), lambda i: (0, 0)),
            pl.BlockSpec((1, D), lambda i: (0, 0)),
        ],
        out_specs=pl.BlockSpec((BM, D), lambda i: (i, 0)),
        out_shape=jax.ShapeDtypeStruct((N, D), _f32),
    )(h, w, b)


def _k1_body(x_ref, w_ref, b_ref, a_ref, b2_ref):
    x = x_ref[...]
    a_ref[...] = jnp.dot(x, w_ref[0], preferred_element_type=_f32) + b_ref[...]
    b2_ref[...] = jnp.dot(x, w_ref[1], preferred_element_type=_f32)


def _k1(x, wstk, b):
    return pl.pallas_call(
        _k1_body,
        grid=(NTC // BM,),
        in_specs=[
            pl.BlockSpec((BM, D), lambda i: (jnp.minimum(i, (N - 1) // BM), 0)),
            pl.BlockSpec((2, D, D), lambda i: (0, 0, 0)),
            pl.BlockSpec((1, D), lambda i: (0, 0)),
        ],
        out_specs=[
            pl.BlockSpec((BM, D), lambda i: (i, 0)),
            pl.BlockSpec((BM, D), lambda i: (i, 0)),
        ],
        out_shape=[
            jax.ShapeDtypeStruct((NTC, D), _f32),
            jax.ShapeDtypeStruct((NTC, D), _f32),
        ],
    )(x, wstk, b)


def _sc_body(a_hbm, b_hbm, src_hbm, dst_hbm, starts_hbm,
             osum, osq, omx, omn, ocnt,
             b2win, arows0, arows1, idxb0, idxb1, dstb0, dstb1, cntb,
             startsv, ssum, ssq, smx, smn, semg0, semg1):
    arowsb = (arows0, arows1)
    idxbb = (idxb0, idxb1)
    dstbb = (dstb0, dstb1)
    semgb = (semg0, semg1)
    wid = lax.axis_index("s") * 2 + lax.axis_index("c")
    pltpu.sync_copy(starts_hbm, startsv)

    zero = jnp.zeros((16,), _f32)
    one = jnp.ones((16,), _f32)
    neg = jnp.full((16,), -1e30, _f32)
    pos = jnp.full((16,), 1e30, _f32)

    G = 8

    def _range(i, _):
        r = wid * NRW + i
        lo = r * R
        sv = startsv[pl.ds(r, 16)]
        s = sv[0]
        e = sv[1]
        pltpu.sync_copy(b_hbm.at[pl.ds(lo, R)], b2win.at[pl.ds(0, R)])

        def _init(ii, _):
            for g in range(G):
                sl = pl.ds(g * 16, 16)
                ssum[ii, sl] = zero
                ssq[ii, sl] = zero
                smx[ii, sl] = neg
                smn[ii, sl] = pos
            cntb[ii, :] = zero
            return 0
        lax.fori_loop(0, R + 1, _init, 0)

        s8 = pl.multiple_of((s // 8) * 8, 8)
        nchunks = (e - s8 + C - 1) // C

        @pl.when(nchunks > 0)
        def _():
            c0 = pl.multiple_of(s8, 8)
            pltpu.sync_copy(src_hbm.at[pl.ds(c0, C)], idxb0)
            pltpu.sync_copy(dst_hbm.at[pl.ds(c0, C)], dstb0)
            pltpu.async_copy(a_hbm.at[idxb0], arows0, semg0)

        def _chunkpair(p, _):
            for b in range(2):
                k = 2 * p + b

                @pl.when(k < nchunks)
                def _(b=b, k=k):
                    arows = arowsb[b]
                    dstb = dstbb[b]
                    pltpu.make_async_copy(
                        a_hbm.at[idxbb[b]], arows, semgb[b]).wait()

                    @pl.when(k + 1 < nchunks)
                    def _():
                        nb = 1 - b
                        c2 = pl.multiple_of(s8 + (k + 1) * C, 8)
                        pltpu.sync_copy(src_hbm.at[pl.ds(c2, C)], idxbb[nb])
                        pltpu.sync_copy(dst_hbm.at[pl.ds(c2, C)], dstbb[nb])
                        pltpu.async_copy(a_hbm.at[idxbb[nb]],
                                         arowsb[nb], semgb[nb])

                    def _edge16(t, _):
                        dvec = dstb[pl.ds(t * 16, 16)] - lo
                        for k16 in range(16):
                            w = dvec[k16]
                            w2 = jnp.where((w >= 0) & (w < R), w, R)
                            j = t * 16 + k16
                            plsc.addupdate(cntb.at[w2, :], one)
                            for g in range(G):
                                sl = pl.ds(g * 16, 16)
                                m = jnp.maximum(
                                    arows[j, sl] + b2win[w2, sl], 0.0)
                                plsc.addupdate(ssum.at[w2, sl], m)
                                plsc.addupdate(ssq.at[w2, sl], m * m)
                                smx[w2, sl] = jnp.maximum(smx[w2, sl], m)
                                smn[w2, sl] = jnp.minimum(smn[w2, sl], m)
                        return 0
                    lax.fori_loop(0, C // 16, _edge16, 0)
            return 0
        lax.fori_loop(0, (nchunks + 1) // 2, _chunkpair, 0)

        pltpu.sync_copy(ssum.at[pl.ds(0, R)], osum.at[pl.ds(lo, R)])
        pltpu.sync_copy(ssq.at[pl.ds(0, R)], osq.at[pl.ds(lo, R)])
        pltpu.sync_copy(smx.at[pl.ds(0, R)], omx.at[pl.ds(lo, R)])
        pltpu.sync_copy(smn.at[pl.ds(0, R)], omn.at[pl.ds(lo, R)])
        pltpu.sync_copy(cntb.at[pl.ds(0, R)], ocnt.at[r])
        return 0

    lax.fori_loop(0, NRW, _range, 0)


@functools.cache
def _sc_call():
    return functools.partial(
        pl.kernel,
        out_type=[
        jax.ShapeDtypeStruct((NPAD, D), _f32),
        jax.ShapeDtypeStruct((NPAD, D), _f32),
        jax.ShapeDtypeStruct((NPAD, D), _f32),
        jax.ShapeDtypeStruct((NPAD, D), _f32),
            jax.ShapeDtypeStruct((NR, R, 16), _f32),
        ],
        mesh=plsc.VectorSubcoreMesh(core_axis_name="c", subcore_axis_name="s"),
        scratch_types=[
            pltpu.VMEM((R + 8, D), _f32),
            pltpu.VMEM((C, D), _f32),
            pltpu.VMEM((C, D), _f32),
            pltpu.VMEM((C,), _i32),
            pltpu.VMEM((C,), _i32),
            pltpu.VMEM((C,), _i32),
            pltpu.VMEM((C,), _i32),
            pltpu.VMEM((R + 8, 16), _f32),
            pltpu.VMEM((NR + 16,), _i32),
            pltpu.VMEM((R + 8, D), _f32),
            pltpu.VMEM((R + 8, D), _f32),
            pltpu.VMEM((R + 8, D), _f32),
            pltpu.VMEM((R + 8, D), _f32),
            pltpu.SemaphoreType.DMA,
            pltpu.SemaphoreType.DMA,
        ],
    )(_sc_body)


def _k2_body(x_ref, sm_ref, sq_ref, mx_ref, mn_ref, cnt_ref, sn_ref,
             wph_ref, w3_ref, b_ref, hn_ref, s1_ref, s2_ref):
    i = pl.program_id(0)
    cnt = cnt_ref[...]
    d = jnp.maximum(cnt, 1.0)
    inv = 1.0 / d
    mean = sm_ref[...] * inv
    var = jnp.maximum(sq_ref[...] * inv - mean * mean, 0.0)
    std = jnp.sqrt(var + 1e-5)
    has = cnt > 0.0
    mx = jnp.where(has, mx_ref[...], 0.0)
    mn = jnp.where(has, mn_ref[...], 0.0)
    agg = jnp.concatenate([mean, mx, mn, std], axis=1)
    logd = jnp.log(d + 1.0)
    amp = logd * (1.0 / AVG_D_LOG)
    att = AVG_D_LOG / logd
    hn = jnp.dot(x_ref[...], wph_ref[...], preferred_element_type=_f32)
    hn = hn + jnp.dot(agg, w3_ref[0], preferred_element_type=_f32)
    hn = hn + amp * jnp.dot(agg, w3_ref[1], preferred_element_type=_f32)
    hn = hn + att * jnp.dot(agg, w3_ref[2], preferred_element_type=_f32)
    hn = (hn + b_ref[...]) * sn_ref[...]
    hn_ref[...] = hn

    @pl.when(i == 0)
    def _():
        s1_ref[...] = jnp.zeros_like(s1_ref)
        s2_ref[...] = jnp.zeros_like(s2_ref)

    s1_ref[...] += jnp.sum(hn, axis=0, keepdims=True)
    s2_ref[...] += jnp.sum(hn * hn, axis=0, keepdims=True)


def _k2(x, sm, sq, mx, mn, cnt, sn, wph, w3, b):
    return pl.pallas_call(
        _k2_body,
        grid=(N // BM2,),
        in_specs=[
            pl.BlockSpec((BM2, D), lambda i: (i, 0)),
            pl.BlockSpec((BM2, D), lambda i: (i, 0)),
            pl.BlockSpec((BM2, D), lambda i: (i, 0)),
            pl.BlockSpec((BM2, D), lambda i: (i, 0)),
            pl.BlockSpec((BM2, D), lambda i: (i, 0)),
            pl.BlockSpec((BM2, 1), lambda i: (i, 0)),
            pl.BlockSpec((BM2, 1), lambda i: (i, 0)),
            pl.BlockSpec((D, D), lambda i: (0, 0)),
            pl.BlockSpec((3, 4 * D, D), lambda i: (0, 0, 0)),
            pl.BlockSpec((1, D), lambda i: (0, 0)),
        ],
        out_specs=[
            pl.BlockSpec((BM2, D), lambda i: (i, 0)),
            pl.BlockSpec((1, D), lambda i: (0, 0)),
            pl.BlockSpec((1, D), lambda i: (0, 0)),
        ],
        out_shape=[
            jax.ShapeDtypeStruct((N, D), _f32),
            jax.ShapeDtypeStruct((1, D), _f32),
            jax.ShapeDtypeStruct((1, D), _f32),
        ],
    )(x, sm, sq, mx, mn, cnt, sn, wph, w3, b)


def _k3_body(hn_ref, x_ref, s1_ref, s2_ref, o_ref):
    mu = s1_ref[...] * (1.0 / N)
    var = s2_ref[...] * (1.0 / N) - mu * mu
    inv = lax.rsqrt(var + 1e-5)
    o_ref[...] = x_ref[...] + jnp.maximum((hn_ref[...] - mu) * inv, 0.0)


def _k3(hn, x, s1, s2):
    return pl.pallas_call(
        _k3_body,
        grid=(N // BM2,),
        in_specs=[
            pl.BlockSpec((BM2, D), lambda i: (i, 0)),
            pl.BlockSpec((BM2, D), lambda i: (i, 0)),
            pl.BlockSpec((1, D), lambda i: (0, 0)),
            pl.BlockSpec((1, D), lambda i: (0, 0)),
        ],
        out_specs=pl.BlockSpec((BM2, D), lambda i: (i, 0)),
        out_shape=jax.ShapeDtypeStruct((N, D), _f32),
    )(hn, x, s1, s2)


def kernel(h, edge_index, e, snorm_n, snorm_e, W_h, b_h, W_pre, b_pre,
           W_post, b_post):
    del e, snorm_e
    src = edge_index[0].astype(_i32)
    dst = edge_index[1].astype(_i32)

    order = jnp.argsort(dst)
    dst_s = dst[order]
    src_s = src[order]
    src_p = jnp.concatenate([src_s, jnp.zeros((EPAD - E,), _i32)])
    dst_p = jnp.concatenate([dst_s, jnp.full((EPAD - E,), SENT, _i32)])
    bounds = jnp.searchsorted(
        dst_s, (jnp.arange(NR + 1, dtype=_i32) * R)).astype(_i32)
    starts = jnp.concatenate([bounds, jnp.full((NR + 16 - (NR + 1),), E, _i32)])

    x = _k0(h, W_h, b_h[None])
    for l in range(L):
        a, b2 = _k1(x, W_pre[l].reshape(2, D, D), b_pre[l][None])
        sm, sq, mx, mn, cnt = _sc_call()(a, b2, src_p, dst_p, starts)
        cnt_n = cnt[:, :, 0].reshape(NPAD)[:N, None]
        hn, s1, s2 = _k2(x, sm[:N], sq[:N], mx[:N], mn[:N], cnt_n, snorm_n,
                         W_post[l, :D], W_post[l, D:].reshape(3, 4 * D, D),
                         b_post[l][None])
        x = _k3(hn, x, s1, s2)
    return x
